# baseline (speedup 1.0000x reference)
import functools

import jax
import jax.numpy as jnp
from jax import lax
from jax.experimental import pallas as pl
from jax.experimental.pallas import tpu as pltpu
from jax.experimental.pallas import tpu_sc as plsc

NC = 2
NS = 16
NW = NC * NS
CHUNK = 128


NBUF = 3


def _build_sc_scatter(npad, d, cd, gather):
  mesh = plsc.VectorSubcoreMesh(core_axis_name="c", subcore_axis_name="s")
  rpt = npad // NS

  scratch = [
      pltpu.VMEM((cd, CHUNK), jnp.int32),
      pltpu.VMEM((cd, CHUNK), jnp.int32),
  ] + [pltpu.VMEM((CHUNK, d), jnp.float32) for _ in range(NBUF)] + [
      pltpu.VMEM_SHARED((npad, d), jnp.float32),
      pltpu.VMEM_SHARED((npad, d), jnp.float32),
  ] + [pltpu.SemaphoreType.DMA for _ in range(2 * NBUF)]

  @functools.partial(
      pl.kernel,
      mesh=mesh,
      out_type=jax.ShapeDtypeStruct((NC, npad, d), jnp.float32),
      scratch_types=scratch,
      compiler_params=pltpu.CompilerParams(use_tc_tiling_on_sc=False),
  )
  def k(u_hbm, src_hbm, dst_hbm, zeros_hbm, out_hbm, idx_s, idx_d, *rest):
    buf = rest[:NBUF]
    ush, acc = rest[NBUF], rest[NBUF + 1]
    gsem = rest[NBUF + 2:NBUF + 2 + NBUF]
    ssem = rest[NBUF + 2 + NBUF:]
    cid = lax.axis_index("c")
    sid = lax.axis_index("s")
    wid = cid * NS + sid
    r0 = sid * rpt
    pltpu.sync_copy(zeros_hbm.at[pl.ds(r0, rpt)], acc.at[pl.ds(r0, rpt)])
    pltpu.sync_copy(dst_hbm.at[wid], idx_d)
    if gather:
      pltpu.sync_copy(u_hbm.at[pl.ds(r0, rpt)], ush.at[pl.ds(r0, rpt)])
      pltpu.sync_copy(src_hbm.at[wid], idx_s)
    else:
      pltpu.sync_copy(u_hbm, buf[0])
    plsc.subcore_barrier()

    if gather:
      for b in range(NBUF):
        pltpu.async_copy(ush.at[idx_s.at[b]], buf[b], gsem[b])

      @pl.loop(0, cd, step=NBUF)
      def _(g):
        for b in range(NBUF):
          pltpu.make_async_copy(ush.at[idx_s.at[g + b]], buf[b],
                                gsem[b]).wait()
          pltpu.async_copy(buf[b], acc.at[idx_d.at[g + b]], ssem[b],
                           add=True)
        for b in range(NBUF):
          pltpu.make_async_copy(buf[b], acc.at[idx_d.at[g + b]],
                                ssem[b]).wait()

          @pl.when(g + NBUF + b < cd)
          def _():
            pltpu.async_copy(ush.at[idx_s.at[g + NBUF + b]], buf[b], gsem[b])
    else:
      for b in range(NBUF):
        pltpu.async_copy(buf[0], acc.at[idx_d.at[b]], ssem[b], add=True)

      @pl.loop(0, cd, step=NBUF)
      def _(g):
        for b in range(NBUF):
          pltpu.make_async_copy(buf[0], acc.at[idx_d.at[g + b]],
                                ssem[b]).wait()

          @pl.when(g + NBUF + b < cd)
          def _():
            pltpu.async_copy(buf[0], acc.at[idx_d.at[g + NBUF + b]],
                             ssem[b], add=True)

    plsc.subcore_barrier()
    pltpu.sync_copy(acc.at[pl.ds(r0, rpt)],
                    out_hbm.at[cid].at[pl.ds(r0, rpt)])

  return k


def _pre_body(n, npad, h, degp, x, w0, dinv_out, u0_out):
  deg = degp[0, :, 0:1] + degp[1, :, 0:1] + 1.0
  dinv_b = jnp.broadcast_to(lax.rsqrt(deg), (npad, h))
  dinv_out[...] = dinv_b
  u0_out[...] = jnp.dot(x[...], w0[...],
                        preferred_element_type=jnp.float32) * dinv_b


def _mid_body(n, npad, sp, u, dinv, bvec, g, bt, wn, out):
  rows = lax.broadcasted_iota(jnp.int32, (npad, 1), 0)
  rmask = rows < n
  h_ = dinv[...] * (sp[0] + sp[1] + u[...]) + bvec[...]
  h_ = jnp.where(rmask, jnp.maximum(h_, 0.0), 0.0)
  mu = jnp.sum(h_, axis=0, keepdims=True) / n
  d0 = jnp.where(rmask, h_ - mu, 0.0)
  var = jnp.sum(d0 * d0, axis=0, keepdims=True) / n
  hn = (h_ - mu) * lax.rsqrt(var + 1e-5) * g[...] + bt[...]
  un = jnp.dot(hn, wn[...], preferred_element_type=jnp.float32) * dinv[...]
  out[...] = jnp.where(rmask, un, 0.0)


def _fin_body(n, npad, gn, c, sp, u, dinv, bvec, batch, yv, fw1, fb1, fw2,
              fb2, minv, midx, loss, maxp_ref):
  rows = lax.broadcasted_iota(jnp.int32, (npad, 1), 0)
  rmask = rows < n
  h_ = dinv[...] * (sp[0] + sp[1] + u[...]) + bvec[...]
  h_ = jnp.where(rmask, jnp.maximum(h_, 0.0), 0.0)

  gids = lax.broadcasted_iota(jnp.int32, (1, gn), 1)
  mf = (batch[...] == gids).astype(jnp.float32)
  counts = jnp.sum(mf, axis=0, keepdims=True)
  sums = lax.dot_general(mf, h_, (((0,), (0,)), ((), ())),
                         preferred_element_type=jnp.float32)
  meanp = sums / jnp.maximum(counts, 1.0).reshape(gn, 1)

  def mx(gi, carry):
    mg = batch[...] == gi
    maxp_ref[pl.ds(gi, 1), :] = jnp.max(
        jnp.where(mg, h_, -jnp.inf), axis=0, keepdims=True)
    return carry

  lax.fori_loop(0, gn, mx, 0)
  maxp = maxp_ref[...]

  gx = jnp.concatenate([meanp, maxp], axis=1)
  hfc = jnp.maximum(
      jnp.dot(gx, fw1[...], preferred_element_type=jnp.float32) + fb1[...],
      0.0)
  logits = jnp.dot(hfc, fw2[...],
                   preferred_element_type=jnp.float32) + fb2[...]
  lmax = jnp.max(logits, axis=1, keepdims=True)
  lse = jnp.log(jnp.sum(jnp.exp(logits - lmax), axis=1, keepdims=True))
  ls = logits - lmax - lse

  ysel = yv[...].reshape(gn, 1) == lax.broadcasted_iota(jnp.int32, (gn, c), 1)
  picked = jnp.sum(jnp.where(ysel, ls, 0.0), axis=1, keepdims=True)
  loss[...] = -jnp.sum(picked, axis=0, keepdims=True) / gn

  sm = jnp.exp(ls)
  mv = jnp.min(sm, axis=0, keepdims=True)
  minv[...] = mv
  ridx = lax.broadcasted_iota(jnp.int32, (gn, c), 0)
  midx[...] = jnp.min(jnp.where(sm == mv, ridx, jnp.int32(2**30)), axis=0,
                      keepdims=True)


def kernel(x, edge_index, batch, y, W0, b0, W1, b1, W2, b2, bn0_g, bn0_b,
           bn1_g, bn1_b, fc1_W, fc1_b, fc2_W, fc2_b):
  n, d_in = x.shape
  h = W0.shape[1]
  gn = y.shape[0]
  c = fc2_W.shape[1]
  e = edge_index.shape[1]

  npad = ((n + NS * 16 - 1) // (NS * 16)) * (NS * 16)
  cd = -(-e // (NW * CHUNK))
  cd += (-cd) % NBUF
  etot = NW * cd * CHUNK

  pad_e = etot - e
  src3 = jnp.concatenate(
      [edge_index[0], jnp.full((pad_e,), n, jnp.int32)]).reshape(NW, cd, CHUNK)
  dst3 = jnp.concatenate(
      [edge_index[1], jnp.full((pad_e,), n, jnp.int32)]).reshape(NW, cd, CHUNK)
  zeros_nd = jnp.zeros((npad, h), jnp.float32)
  zeros_d8 = jnp.zeros((npad, 8), jnp.float32)
  ones_chunk = jnp.ones((CHUNK, 8), jnp.float32)
  x_pad = jnp.pad(x, ((0, npad - n), (0, 0)))
  batch_pad = jnp.pad(batch, (0, npad - n),
                      constant_values=gn).reshape(npad, 1)

  sc_deg = _build_sc_scatter(npad, 8, cd, gather=False)
  sc_agg = _build_sc_scatter(npad, h, cd, gather=True)

  def tc(body, out_shape, *args):
    return pl.pallas_call(body, out_shape=out_shape)(*args)

  degp = sc_deg(ones_chunk, src3, dst3, zeros_d8)
  nd = jax.ShapeDtypeStruct((npad, h), jnp.float32)
  dinv_b, u0 = tc(functools.partial(_pre_body, n, npad, h), (nd, nd),
                  degp, x_pad, W0)

  sp0 = sc_agg(u0, src3, dst3, zeros_nd)
  u1 = tc(functools.partial(_mid_body, n, npad), nd,
          sp0, u0, dinv_b, b0.reshape(1, h), bn0_g.reshape(1, h),
          bn0_b.reshape(1, h), W1)

  sp1 = sc_agg(u1, src3, dst3, zeros_nd)
  u2 = tc(functools.partial(_mid_body, n, npad), nd,
          sp1, u1, dinv_b, b1.reshape(1, h), bn1_g.reshape(1, h),
          bn1_b.reshape(1, h), W2)

  sp2 = sc_agg(u2, src3, dst3, zeros_nd)
  minv, midx, loss = pl.pallas_call(
      functools.partial(_fin_body, n, npad, gn, c),
      out_shape=(jax.ShapeDtypeStruct((1, c), jnp.float32),
                 jax.ShapeDtypeStruct((1, c), jnp.int32),
                 jax.ShapeDtypeStruct((1, 1), jnp.float32)),
      scratch_shapes=[pltpu.VMEM((gn, h), jnp.float32)],
  )(sp2, u2, dinv_b, b2.reshape(1, h), batch_pad, y.reshape(1, gn),
    fc1_W, fc1_b.reshape(1, h), fc2_W, fc2_b.reshape(1, c))

  return (minv.reshape(c), midx.reshape(c), loss.reshape(()))


# speedup vs baseline: 1.1524x; 1.1524x over previous
import functools

import jax
import jax.numpy as jnp
from jax import lax
from jax.experimental import pallas as pl
from jax.experimental.pallas import tpu as pltpu
from jax.experimental.pallas import tpu_sc as plsc

NC = 2
NS = 16
NW = NC * NS
CHUNK = 128


NBUF = 4


def _build_sc_scatter(npad, d, cd, gather):
  mesh = plsc.VectorSubcoreMesh(core_axis_name="c", subcore_axis_name="s")
  rpt = npad // NS

  nbuf = 2 if gather else 1
  nsem = 2 if gather else NBUF
  scratch = [
      pltpu.VMEM((cd, CHUNK), jnp.int32),
      pltpu.VMEM((cd, CHUNK), jnp.int32),
  ] + [pltpu.VMEM((CHUNK, d), jnp.float32) for _ in range(nbuf)] + [
      pltpu.VMEM_SHARED((npad, d), jnp.float32),
      pltpu.VMEM_SHARED((npad, d), jnp.float32),
  ] + [pltpu.SemaphoreType.DMA for _ in range(nsem)]

  @functools.partial(
      pl.kernel,
      mesh=mesh,
      out_type=jax.ShapeDtypeStruct((NC, npad, d), jnp.float32),
      scratch_types=scratch,
      compiler_params=pltpu.CompilerParams(use_tc_tiling_on_sc=False),
  )
  def k(u_hbm, src_hbm, dst_hbm, zeros_hbm, out_hbm, idx_s, idx_d, *rest):
    buf = rest[:nbuf]
    ush, acc = rest[nbuf], rest[nbuf + 1]
    sem = rest[nbuf + 2:]
    cid = lax.axis_index("c")
    sid = lax.axis_index("s")
    wid = cid * NS + sid
    r0 = sid * rpt
    pltpu.sync_copy(zeros_hbm.at[pl.ds(r0, rpt)], acc.at[pl.ds(r0, rpt)])
    pltpu.sync_copy(dst_hbm.at[wid], idx_d)
    if gather:
      pltpu.sync_copy(u_hbm.at[pl.ds(r0, rpt)], ush.at[pl.ds(r0, rpt)])
      pltpu.sync_copy(src_hbm.at[wid], idx_s)
    else:
      pltpu.sync_copy(u_hbm, buf[0])
    plsc.subcore_barrier()

    if gather:
      buf0, buf1 = buf
      sem0, sem1 = sem
      pltpu.async_copy(ush.at[idx_s.at[0]], buf0, sem0)
      pltpu.async_copy(ush.at[idx_s.at[1]], buf1, sem1)

      @pl.loop(0, cd - 2, step=2)
      def _(g):
        pltpu.make_async_copy(ush.at[idx_s.at[g]], buf0, sem0).wait()
        pltpu.sync_copy(buf0, acc.at[idx_d.at[g]], add=True)
        pltpu.async_copy(ush.at[idx_s.at[g + 2]], buf0, sem0)
        pltpu.make_async_copy(ush.at[idx_s.at[g + 1]], buf1, sem1).wait()
        pltpu.sync_copy(buf1, acc.at[idx_d.at[g + 1]], add=True)
        pltpu.async_copy(ush.at[idx_s.at[g + 3]], buf1, sem1)

      pltpu.make_async_copy(ush.at[idx_s.at[cd - 2]], buf0, sem0).wait()
      pltpu.sync_copy(buf0, acc.at[idx_d.at[cd - 2]], add=True)
      pltpu.make_async_copy(ush.at[idx_s.at[cd - 1]], buf1, sem1).wait()
      pltpu.sync_copy(buf1, acc.at[idx_d.at[cd - 1]], add=True)
    else:
      for b in range(NBUF):
        pltpu.async_copy(buf[0], acc.at[idx_d.at[b]], sem[b], add=True)

      @pl.loop(0, cd, step=NBUF)
      def _(g):
        for b in range(NBUF):
          pltpu.make_async_copy(buf[0], acc.at[idx_d.at[g + b]],
                                sem[b]).wait()

          @pl.when(g + NBUF + b < cd)
          def _():
            pltpu.async_copy(buf[0], acc.at[idx_d.at[g + NBUF + b]],
                             sem[b], add=True)

    plsc.subcore_barrier()
    pltpu.sync_copy(acc.at[pl.ds(r0, rpt)],
                    out_hbm.at[cid].at[pl.ds(r0, rpt)])

  return k


def _pre_body(n, npad, h, degp, x, w0, dinv_out, u0_out):
  deg = degp[0, :, 0:1] + degp[1, :, 0:1] + 1.0
  dinv_b = jnp.broadcast_to(lax.rsqrt(deg), (npad, h))
  dinv_out[...] = dinv_b
  u0_out[...] = jnp.dot(x[...], w0[...],
                        preferred_element_type=jnp.float32) * dinv_b


def _mid_body(n, npad, sp, u, dinv, bvec, g, bt, wn, out):
  rows = lax.broadcasted_iota(jnp.int32, (npad, 1), 0)
  rmask = rows < n
  h_ = dinv[...] * (sp[0] + sp[1] + u[...]) + bvec[...]
  h_ = jnp.where(rmask, jnp.maximum(h_, 0.0), 0.0)
  mu = jnp.sum(h_, axis=0, keepdims=True) / n
  d0 = jnp.where(rmask, h_ - mu, 0.0)
  var = jnp.sum(d0 * d0, axis=0, keepdims=True) / n
  hn = (h_ - mu) * lax.rsqrt(var + 1e-5) * g[...] + bt[...]
  un = jnp.dot(hn, wn[...], preferred_element_type=jnp.float32) * dinv[...]
  out[...] = jnp.where(rmask, un, 0.0)


def _fin_body(n, npad, gn, c, sp, u, dinv, bvec, batch, yv, fw1, fb1, fw2,
              fb2, minv, midx, loss, maxp_ref):
  rows = lax.broadcasted_iota(jnp.int32, (npad, 1), 0)
  rmask = rows < n
  h_ = dinv[...] * (sp[0] + sp[1] + u[...]) + bvec[...]
  h_ = jnp.where(rmask, jnp.maximum(h_, 0.0), 0.0)

  gids = lax.broadcasted_iota(jnp.int32, (1, gn), 1)
  mf = (batch[...] == gids).astype(jnp.float32)
  counts = jnp.sum(mf, axis=0, keepdims=True)
  sums = lax.dot_general(mf, h_, (((0,), (0,)), ((), ())),
                         preferred_element_type=jnp.float32)
  meanp = sums / jnp.maximum(counts, 1.0).reshape(gn, 1)

  def mx(gi, carry):
    mg = batch[...] == gi
    maxp_ref[pl.ds(gi, 1), :] = jnp.max(
        jnp.where(mg, h_, -jnp.inf), axis=0, keepdims=True)
    return carry

  lax.fori_loop(0, gn, mx, 0)
  maxp = maxp_ref[...]

  gx = jnp.concatenate([meanp, maxp], axis=1)
  hfc = jnp.maximum(
      jnp.dot(gx, fw1[...], preferred_element_type=jnp.float32) + fb1[...],
      0.0)
  logits = jnp.dot(hfc, fw2[...],
                   preferred_element_type=jnp.float32) + fb2[...]
  lmax = jnp.max(logits, axis=1, keepdims=True)
  lse = jnp.log(jnp.sum(jnp.exp(logits - lmax), axis=1, keepdims=True))
  ls = logits - lmax - lse

  ysel = yv[...].reshape(gn, 1) == lax.broadcasted_iota(jnp.int32, (gn, c), 1)
  picked = jnp.sum(jnp.where(ysel, ls, 0.0), axis=1, keepdims=True)
  loss[...] = -jnp.sum(picked, axis=0, keepdims=True) / gn

  sm = jnp.exp(ls)
  mv = jnp.min(sm, axis=0, keepdims=True)
  minv[...] = mv
  ridx = lax.broadcasted_iota(jnp.int32, (gn, c), 0)
  midx[...] = jnp.min(jnp.where(sm == mv, ridx, jnp.int32(2**30)), axis=0,
                      keepdims=True)


def kernel(x, edge_index, batch, y, W0, b0, W1, b1, W2, b2, bn0_g, bn0_b,
           bn1_g, bn1_b, fc1_W, fc1_b, fc2_W, fc2_b):
  n, d_in = x.shape
  h = W0.shape[1]
  gn = y.shape[0]
  c = fc2_W.shape[1]
  e = edge_index.shape[1]

  npad = ((n + NS * 16 - 1) // (NS * 16)) * (NS * 16)
  cd = -(-e // (NW * CHUNK))
  cd += (-cd) % NBUF
  etot = NW * cd * CHUNK

  pad_e = etot - e
  src3 = jnp.concatenate(
      [edge_index[0], jnp.full((pad_e,), n, jnp.int32)]).reshape(NW, cd, CHUNK)
  dst3 = jnp.concatenate(
      [edge_index[1], jnp.full((pad_e,), n, jnp.int32)]).reshape(NW, cd, CHUNK)
  zeros_nd = jnp.zeros((npad, h), jnp.float32)
  zeros_d8 = jnp.zeros((npad, 8), jnp.float32)
  ones_chunk = jnp.ones((CHUNK, 8), jnp.float32)
  x_pad = jnp.pad(x, ((0, npad - n), (0, 0)))
  batch_pad = jnp.pad(batch, (0, npad - n),
                      constant_values=gn).reshape(npad, 1)

  sc_deg = _build_sc_scatter(npad, 8, cd, gather=False)
  sc_agg = _build_sc_scatter(npad, h, cd, gather=True)

  def tc(body, out_shape, *args):
    return pl.pallas_call(body, out_shape=out_shape)(*args)

  degp = sc_deg(ones_chunk, src3, dst3, zeros_d8)
  nd = jax.ShapeDtypeStruct((npad, h), jnp.float32)
  dinv_b, u0 = tc(functools.partial(_pre_body, n, npad, h), (nd, nd),
                  degp, x_pad, W0)

  sp0 = sc_agg(u0, src3, dst3, zeros_nd)
  u1 = tc(functools.partial(_mid_body, n, npad), nd,
          sp0, u0, dinv_b, b0.reshape(1, h), bn0_g.reshape(1, h),
          bn0_b.reshape(1, h), W1)

  sp1 = sc_agg(u1, src3, dst3, zeros_nd)
  u2 = tc(functools.partial(_mid_body, n, npad), nd,
          sp1, u1, dinv_b, b1.reshape(1, h), bn1_g.reshape(1, h),
          bn1_b.reshape(1, h), W2)

  sp2 = sc_agg(u2, src3, dst3, zeros_nd)
  minv, midx, loss = pl.pallas_call(
      functools.partial(_fin_body, n, npad, gn, c),
      out_shape=(jax.ShapeDtypeStruct((1, c), jnp.float32),
                 jax.ShapeDtypeStruct((1, c), jnp.int32),
                 jax.ShapeDtypeStruct((1, 1), jnp.float32)),
      scratch_shapes=[pltpu.VMEM((gn, h), jnp.float32)],
  )(sp2, u2, dinv_b, b2.reshape(1, h), batch_pad, y.reshape(1, gn),
    fc1_W, fc1_b.reshape(1, h), fc2_W, fc2_b.reshape(1, c))

  return (minv.reshape(c), midx.reshape(c), loss.reshape(()))
